# baseline (speedup 1.0000x reference)
import functools

import jax
import jax.numpy as jnp
from jax import lax
from jax.experimental import pallas as pl
from jax.experimental.pallas import tpu as pltpu
from jax.experimental.pallas import tpu_sc as plsc

N_TOK = 32768
DIM = 128
HEADS = 8
HDIM = 16
NUM_WIN = 1024

NW = 32
W_PER = NUM_WIN // NW
CH = 128
OFF_PAD = 1056


def _qkv_body(x_ref, w_ref, q_ref, k_ref, v_ref):
    acc = jnp.dot(x_ref[...], w_ref[...], preferred_element_type=jnp.float32)
    q_ref[...] = jnp.maximum(acc[:, 0:DIM], 0.0)
    k_ref[...] = jnp.maximum(acc[:, DIM:2 * DIM], 0.0)
    v_ref[...] = acc[:, 2 * DIM:3 * DIM]


def _qkv_call(x, w_qkv):
    blk = 2048
    grid = (N_TOK // blk,)
    out = jax.ShapeDtypeStruct((N_TOK, DIM), jnp.float32)
    return pl.pallas_call(
        _qkv_body,
        grid=grid,
        in_specs=[
            pl.BlockSpec((blk, DIM), lambda i: (i, 0)),
            pl.BlockSpec((DIM, 3 * DIM), lambda i: (0, 0)),
        ],
        out_specs=[
            pl.BlockSpec((blk, DIM), lambda i: (i, 0)),
            pl.BlockSpec((blk, DIM), lambda i: (i, 0)),
            pl.BlockSpec((blk, DIM), lambda i: (i, 0)),
        ],
        out_shape=[out, out, out],
    )(x, w_qkv)


def _proj_body(y_ref, w_ref, b_ref, o_ref):
    o_ref[...] = (
        jnp.dot(y_ref[...], w_ref[...], preferred_element_type=jnp.float32)
        + b_ref[...]
    )


def _proj_call(y, w_proj, b_proj):
    blk = 2048
    grid = (N_TOK // blk,)
    return pl.pallas_call(
        _proj_body,
        grid=grid,
        in_specs=[
            pl.BlockSpec((blk, DIM), lambda i: (i, 0)),
            pl.BlockSpec((DIM, DIM), lambda i: (0, 0)),
            pl.BlockSpec((1, DIM), lambda i: (0, 0)),
        ],
        out_specs=pl.BlockSpec((blk, DIM), lambda i: (i, 0)),
        out_shape=jax.ShapeDtypeStruct((N_TOK, DIM), jnp.float32),
    )(y, w_proj, b_proj.reshape(1, DIM))


@functools.partial(
    pl.kernel,
    out_type=jax.ShapeDtypeStruct((N_TOK * DIM,), jnp.float32),
    mesh=plsc.VectorSubcoreMesh(core_axis_name="c", subcore_axis_name="s"),
    scratch_types=[
        pltpu.VMEM((W_PER * HEADS * HDIM * HDIM,), jnp.float32),
        pltpu.VMEM((W_PER * HEADS * HDIM,), jnp.float32),
        pltpu.VMEM((CH * DIM,), jnp.float32),
        pltpu.VMEM((CH * DIM,), jnp.float32),
        pltpu.VMEM((64,), jnp.int32),
        pltpu.VMEM((HEADS * CH,), jnp.float32),
        pltpu.SemaphoreType.DMA,
        pltpu.SemaphoreType.DMA,
    ],
    compiler_params=pltpu.CompilerParams(needs_layout_passes=False),
)
def _sla_sc(q_hbm, k_hbm, v_hbm, off_hbm, y_hbm,
            kv_s, s_s, buf_a, buf_b, off_s, zi_s, sem_a, sem_b):
    wid = lax.axis_index("s") * 2 + lax.axis_index("c")
    w0 = wid * W_PER

    pltpu.sync_copy(off_hbm.at[pl.ds(w0, 48)], off_s.at[pl.ds(0, 48)])
    tok_start = off_s[pl.ds(0, 16)][0]
    tok_end = off_s[pl.ds(W_PER, 16)][0]

    zero = jnp.zeros((HDIM,), jnp.float32)

    def _zero_kv(i, _):
        kv_s[pl.ds(i * HDIM, HDIM)] = zero
        return 0

    lax.fori_loop(0, W_PER * HEADS * HDIM, _zero_kv, 0)

    def _zero_s(i, _):
        s_s[pl.ds(i * HDIM, HDIM)] = zero
        return 0

    lax.fori_loop(0, W_PER * HEADS, _zero_s, 0)

    n_chunks = (tok_end - tok_start + CH - 1) // CH

    off_lo = off_s[pl.ds(1, 16)]
    off_hi = off_s[pl.ds(17, 16)]

    def _win_of(t):
        tv = jnp.full((16,), t, jnp.int32)
        n_lo = plsc.all_reduce_population_count(off_lo <= tv)
        n_hi = plsc.all_reduce_population_count(off_hi <= tv)
        return n_lo[0] + n_hi[0]

    def _chunk1(ci, w_cur):
        tok_cur = tok_start + ci * CH
        base = jnp.minimum(tok_cur, N_TOK - CH)
        dl = tok_cur - base
        nt = jnp.minimum(CH, tok_end - tok_cur)
        cp_k = pltpu.async_copy(
            k_hbm.at[pl.ds(base * DIM, CH * DIM)], buf_a, sem_a)
        cp_v = pltpu.async_copy(
            v_hbm.at[pl.ds(base * DIM, CH * DIM)], buf_b, sem_b)
        cp_k.wait()
        cp_v.wait()

        def _tok(j, _):
            r = (dl + j) * DIM
            w_cur = _win_of(tok_cur + j)
            for h in range(HEADS):
                kvec = buf_a[pl.ds(r + h * HDIM, HDIM)]
                vvec = buf_b[pl.ds(r + h * HDIM, HDIM)]
                plsc.addupdate(
                    s_s.at[pl.ds((w_cur * HEADS + h) * HDIM, HDIM)], kvec)
                rb = (w_cur * (HEADS * HDIM) + h * HDIM) * HDIM
                for c1 in range(HDIM):
                    plsc.addupdate(
                        kv_s.at[pl.ds(rb + c1 * HDIM, HDIM)],
                        jnp.full((HDIM,), kvec[c1]) * vvec)
            return 0

        lax.fori_loop(0, nt, _tok, 0)
        return 0

    lax.fori_loop(0, n_chunks, _chunk1, 0)

    def _chunk2(ci, w_cur):
        tok_cur = tok_start + ci * CH
        base = jnp.minimum(tok_cur, N_TOK - CH)
        dl = tok_cur - base
        nt = jnp.minimum(CH, tok_end - tok_cur)
        pltpu.async_copy(
            q_hbm.at[pl.ds(base * DIM, CH * DIM)], buf_a, sem_a).wait()

        w_first = _win_of(tok_cur)
        w_last = _win_of(tok_cur + nt - 1)

        def _win(w, _):
            t_lo = jnp.maximum(off_s[pl.ds(w, 16)][0], tok_cur)
            t_hi = jnp.minimum(off_s[pl.ds(w + 1, 16)][0], tok_cur + nt)
            for h in range(HEADS):
                rb = (w * (HEADS * HDIM) + h * HDIM) * HDIM
                kvr = [kv_s[pl.ds(rb + c1 * HDIM, HDIM)]
                       for c1 in range(HDIM)]
                svec = s_s[pl.ds((w * HEADS + h) * HDIM, HDIM)]

                @plsc.parallel_loop(t_lo, t_hi, 1, unroll=4)
                def _tok(t, h=h, kvr=kvr, svec=svec):
                    r = (t - base) * DIM
                    qvec = buf_a[pl.ds(r + h * HDIM, HDIM)]
                    z = jnp.sum(qvec * svec) + 1e-6
                    vs = [jnp.full((HDIM,), qvec[c1]) * kvr[c1]
                          for c1 in range(HDIM)]
                    while len(vs) > 1:
                        vs = [vs[i] + vs[i + 1]
                              for i in range(0, len(vs), 2)]
                    buf_b[pl.ds(r + h * HDIM, HDIM)] = vs[0] / z
            return 0

        lax.fori_loop(w_first, w_last + 1, _win, 0)

        rem = nt
        src = dl
        dst = tok_cur
        sz = CH
        while sz >= 1:
            cond = rem >= sz
            sz_now = sz

            @pl.when(cond)
            def _copy(src=src, dst=dst, sz_now=sz_now):
                pltpu.sync_copy(
                    buf_b.at[pl.ds(src * DIM, sz_now * DIM)],
                    y_hbm.at[pl.ds(dst * DIM, sz_now * DIM)])

            step = jnp.where(cond, sz, 0)
            src = src + step
            dst = dst + step
            rem = rem - step
            sz //= 2
        return 0

    lax.fori_loop(0, n_chunks, _chunk2, 0)


@jax.jit
def _run(x, offsets, w_qkv, w_proj, b_proj):
    q, k, v = _qkv_call(x, w_qkv)
    off_ext = jnp.concatenate([
        offsets.astype(jnp.int32),
        jnp.full((OFF_PAD - NUM_WIN,), N_TOK, jnp.int32),
    ])
    y = _sla_sc(q.reshape(-1), k.reshape(-1), v.reshape(-1), off_ext)
    return _proj_call(y.reshape(N_TOK, DIM), w_proj, b_proj)


def kernel(x, offsets, counts, batch_win_inds, W_qkv, W_proj, b_proj):
    return _run(x, offsets, W_qkv, W_proj, b_proj)


# speedup vs baseline: 1.3480x; 1.3480x over previous
import functools

import jax
import jax.numpy as jnp
from jax import lax
from jax.experimental import pallas as pl
from jax.experimental.pallas import tpu as pltpu
from jax.experimental.pallas import tpu_sc as plsc

N_TOK = 32768
DIM = 128
HEADS = 8
HDIM = 16
NUM_WIN = 1024

NW = 32
W_PER = NUM_WIN // NW
CH = 128
OFF_PAD = 1056


def _qkv_body(x_ref, w_ref, q_ref, k_ref, v_ref):
    acc = jnp.dot(x_ref[...], w_ref[...], preferred_element_type=jnp.float32)
    q_ref[...] = jnp.maximum(acc[:, 0:DIM], 0.0)
    k_ref[...] = jnp.maximum(acc[:, DIM:2 * DIM], 0.0)
    v_ref[...] = acc[:, 2 * DIM:3 * DIM]


def _qkv_call(x, w_qkv):
    blk = 2048
    grid = (N_TOK // blk,)
    out = jax.ShapeDtypeStruct((N_TOK, DIM), jnp.float32)
    return pl.pallas_call(
        _qkv_body,
        grid=grid,
        in_specs=[
            pl.BlockSpec((blk, DIM), lambda i: (i, 0)),
            pl.BlockSpec((DIM, 3 * DIM), lambda i: (0, 0)),
        ],
        out_specs=[
            pl.BlockSpec((blk, DIM), lambda i: (i, 0)),
            pl.BlockSpec((blk, DIM), lambda i: (i, 0)),
            pl.BlockSpec((blk, DIM), lambda i: (i, 0)),
        ],
        out_shape=[out, out, out],
    )(x, w_qkv)


def _proj_body(y_ref, w_ref, b_ref, o_ref):
    o_ref[...] = (
        jnp.dot(y_ref[...], w_ref[...], preferred_element_type=jnp.float32)
        + b_ref[...]
    )


def _proj_call(y, w_proj, b_proj):
    blk = 2048
    grid = (N_TOK // blk,)
    return pl.pallas_call(
        _proj_body,
        grid=grid,
        in_specs=[
            pl.BlockSpec((blk, DIM), lambda i: (i, 0)),
            pl.BlockSpec((DIM, DIM), lambda i: (0, 0)),
            pl.BlockSpec((1, DIM), lambda i: (0, 0)),
        ],
        out_specs=pl.BlockSpec((blk, DIM), lambda i: (i, 0)),
        out_shape=jax.ShapeDtypeStruct((N_TOK, DIM), jnp.float32),
    )(y, w_proj, b_proj.reshape(1, DIM))


@functools.partial(
    pl.kernel,
    out_type=jax.ShapeDtypeStruct((N_TOK * DIM,), jnp.float32),
    mesh=plsc.VectorSubcoreMesh(core_axis_name="c", subcore_axis_name="s"),
    scratch_types=[
        pltpu.VMEM((W_PER * HEADS * HDIM * HDIM,), jnp.float32),
        pltpu.VMEM((W_PER * HEADS * HDIM,), jnp.float32),
        pltpu.VMEM((CH * DIM,), jnp.float32),
        pltpu.VMEM((CH * DIM,), jnp.float32),
        pltpu.VMEM((64,), jnp.int32),
        pltpu.VMEM((HEADS * CH,), jnp.float32),
        pltpu.SemaphoreType.DMA,
        pltpu.SemaphoreType.DMA,
    ],
    compiler_params=pltpu.CompilerParams(needs_layout_passes=False),
)
def _sla_sc(q_hbm, k_hbm, v_hbm, off_hbm, y_hbm,
            kv_s, s_s, buf_a, buf_b, off_s, zi_s, sem_a, sem_b):
    wid = lax.axis_index("s") * 2 + lax.axis_index("c")
    w0 = wid * W_PER

    pltpu.sync_copy(off_hbm.at[pl.ds(w0, 48)], off_s.at[pl.ds(0, 48)])
    tok_start = off_s[pl.ds(0, 16)][0]
    tok_end = off_s[pl.ds(W_PER, 16)][0]

    zero = jnp.zeros((HDIM,), jnp.float32)

    def _zero_kv(i, _):
        kv_s[pl.ds(i * HDIM, HDIM)] = zero
        return 0

    lax.fori_loop(0, W_PER * HEADS * HDIM, _zero_kv, 0)

    def _zero_s(i, _):
        s_s[pl.ds(i * HDIM, HDIM)] = zero
        return 0

    lax.fori_loop(0, W_PER * HEADS, _zero_s, 0)

    n_chunks = (tok_end - tok_start + CH - 1) // CH

    off_lo = off_s[pl.ds(1, 16)]
    off_hi = off_s[pl.ds(17, 16)]

    def _win_of(t):
        tv = jnp.full((16,), t, jnp.int32)
        n_lo = plsc.all_reduce_population_count(off_lo <= tv)
        n_hi = plsc.all_reduce_population_count(off_hi <= tv)
        return n_lo[0] + n_hi[0]

    def _chunk1(ci, w_cur):
        tok_cur = tok_start + ci * CH
        base = jnp.minimum(tok_cur, N_TOK - CH)
        dl = tok_cur - base
        nt = jnp.minimum(CH, tok_end - tok_cur)
        cp_k = pltpu.async_copy(
            k_hbm.at[pl.ds(base * DIM, CH * DIM)], buf_a, sem_a)
        cp_v = pltpu.async_copy(
            v_hbm.at[pl.ds(base * DIM, CH * DIM)], buf_b, sem_b)
        cp_k.wait()
        cp_v.wait()

        w_first = _win_of(tok_cur)
        w_last = _win_of(tok_cur + nt - 1)

        def _win(w, _):
            t_lo = jnp.maximum(off_s[pl.ds(w, 16)][0], tok_cur)
            t_hi = jnp.minimum(off_s[pl.ds(w + 1, 16)][0], tok_cur + nt)
            for h in range(HEADS):
                rb = (w * (HEADS * HDIM) + h * HDIM) * HDIM
                sb = (w * HEADS + h) * HDIM
                init = tuple(
                    kv_s[pl.ds(rb + c1 * HDIM, HDIM)]
                    for c1 in range(HDIM)) + (s_s[pl.ds(sb, HDIM)],)

                @plsc.parallel_loop(t_lo, t_hi, 1, unroll=2, carry=init)
                def _tok(t, acc, h=h):
                    r = (t - base) * DIM
                    kvec = buf_a[pl.ds(r + h * HDIM, HDIM)]
                    vvec = buf_b[pl.ds(r + h * HDIM, HDIM)]
                    return tuple(
                        acc[c1] + jnp.full((HDIM,), kvec[c1]) * vvec
                        for c1 in range(HDIM)) + (acc[HDIM] + kvec,)

                for c1 in range(HDIM):
                    kv_s[pl.ds(rb + c1 * HDIM, HDIM)] = _tok[c1]
                s_s[pl.ds(sb, HDIM)] = _tok[HDIM]
            return 0

        lax.fori_loop(w_first, w_last + 1, _win, 0)
        return 0

    lax.fori_loop(0, n_chunks, _chunk1, 0)

    def _chunk2(ci, w_cur):
        tok_cur = tok_start + ci * CH
        base = jnp.minimum(tok_cur, N_TOK - CH)
        dl = tok_cur - base
        nt = jnp.minimum(CH, tok_end - tok_cur)
        pltpu.async_copy(
            q_hbm.at[pl.ds(base * DIM, CH * DIM)], buf_a, sem_a).wait()

        w_first = _win_of(tok_cur)
        w_last = _win_of(tok_cur + nt - 1)

        def _win(w, _):
            t_lo = jnp.maximum(off_s[pl.ds(w, 16)][0], tok_cur)
            t_hi = jnp.minimum(off_s[pl.ds(w + 1, 16)][0], tok_cur + nt)
            for h in range(HEADS):
                rb = (w * (HEADS * HDIM) + h * HDIM) * HDIM
                kvr = [kv_s[pl.ds(rb + c1 * HDIM, HDIM)]
                       for c1 in range(HDIM)]
                svec = s_s[pl.ds((w * HEADS + h) * HDIM, HDIM)]

                @plsc.parallel_loop(t_lo, t_hi, 1, unroll=2)
                def _tok(t, h=h, kvr=kvr, svec=svec):
                    r = (t - base) * DIM
                    qvec = buf_a[pl.ds(r + h * HDIM, HDIM)]
                    z = jnp.sum(qvec * svec) + 1e-6
                    vs = [jnp.full((HDIM,), qvec[c1]) * kvr[c1]
                          for c1 in range(HDIM)]
                    while len(vs) > 1:
                        vs = [vs[i] + vs[i + 1]
                              for i in range(0, len(vs), 2)]
                    buf_b[pl.ds(r + h * HDIM, HDIM)] = vs[0] / z
            return 0

        lax.fori_loop(w_first, w_last + 1, _win, 0)

        rem = nt
        src = dl
        dst = tok_cur
        sz = CH
        while sz >= 1:
            cond = rem >= sz
            sz_now = sz

            @pl.when(cond)
            def _copy(src=src, dst=dst, sz_now=sz_now):
                pltpu.sync_copy(
                    buf_b.at[pl.ds(src * DIM, sz_now * DIM)],
                    y_hbm.at[pl.ds(dst * DIM, sz_now * DIM)])

            step = jnp.where(cond, sz, 0)
            src = src + step
            dst = dst + step
            rem = rem - step
            sz //= 2
        return 0

    lax.fori_loop(0, n_chunks, _chunk2, 0)


@jax.jit
def _run(x, offsets, w_qkv, w_proj, b_proj):
    q, k, v = _qkv_call(x, w_qkv)
    off_ext = jnp.concatenate([
        offsets.astype(jnp.int32),
        jnp.full((OFF_PAD - NUM_WIN,), N_TOK, jnp.int32),
    ])
    y = _sla_sc(q.reshape(-1), k.reshape(-1), v.reshape(-1), off_ext)
    return _proj_call(y.reshape(N_TOK, DIM), w_proj, b_proj)


def kernel(x, offsets, counts, batch_win_inds, W_qkv, W_proj, b_proj):
    return _run(x, offsets, W_qkv, W_proj, b_proj)
